# baseline (speedup 1.0000x reference)
import numpy as np
import jax
import jax.numpy as jnp
from jax import lax
from jax.experimental import pallas as pl
from jax.experimental.pallas import tpu as pltpu
from jax.experimental.pallas import tpu_sc as plsc

N = 10000
E = 320000
DIM = 128
G = 1024
BLK = 1024
NBLK = (N + BLK - 1) // BLK

NC = 2
NS = 16
NW = NC * NS
LANES = 16
EPT = E // NW
BATCH = 96
CAP = EPT + 2 * BATCH
SROWS = G + LANES
CW = 16
ZR = SROWS // NS

_half = 32
_om = 1.0 / (10000.0 ** (np.arange(_half, dtype=np.float32) / _half))
_OM2 = np.concatenate([_om, _om, _om, _om]).reshape(1, DIM).astype(np.float32)
_SEL = np.concatenate([np.zeros(64), np.ones(64)]).reshape(1, DIM).astype(np.float32)
_PH = np.concatenate([np.zeros(32), np.full(32, np.pi / 2),
                      np.zeros(32), np.full(32, np.pi / 2)]).reshape(1, DIM)
_PH = _PH.astype(np.float32)


def _silu(v):
    return v * (1.0 / (1.0 + jnp.exp(-v)))


def _prep_body(x_ref, pos_ref, om_ref, sel_ref, ph_ref,
               pW_ref, pb_ref, w1_ref, b1_ref, w2_ref, b2_ref,
               wa_ref, wb_ref, gb1_ref,
               a_ref, b_ref, hg_ref, h_s):
    pid = pl.program_id(0)
    x = x_ref[...]
    pos = pos_ref[...]
    sel = sel_ref[...]
    posc = pos[:, 0:1] * (1.0 - sel) + pos[:, 1:2] * sel
    pe = jnp.sin(posc * om_ref[...] + ph_ref[...])

    @pl.when(pid < 1)
    def _():
        t = _silu(jnp.dot(pe, w1_ref[...], preferred_element_type=jnp.float32)
                  + b1_ref[...])
        u = (jnp.dot(t, w2_ref[...], preferred_element_type=jnp.float32)
             + b2_ref[...])
        h = u + pe
        h_s[...] = h
        b_ref[...] = (jnp.dot(h, wb_ref[...], preferred_element_type=jnp.float32)
                      + gb1_ref[...])
        hg_ref[...] = h

    @pl.when(pid >= 1)
    def _():
        h_s[...] = (x[:, 0:1] * pW_ref[0:1, :] + x[:, 1:2] * pW_ref[1:2, :]
                    + x[:, 2:3] * pW_ref[2:3, :] + pb_ref[...]) + pe

    a_ref[...] = jnp.dot(h_s[...], wa_ref[...],
                         preferred_element_type=jnp.float32)


def _prep(x, pos, proj_W, proj_b, pm_W1, pm_b1, pm_W2, pm_b2, W1a, W1b, g_bm1):
    full = pl.BlockSpec((1, DIM), lambda i: (0, 0))
    mat = pl.BlockSpec((DIM, DIM), lambda i: (0, 0))
    return pl.pallas_call(
        _prep_body,
        grid=(NBLK,),
        in_specs=[
            pl.BlockSpec((BLK, 3), lambda i: (i, 0)),
            pl.BlockSpec((BLK, 2), lambda i: (i, 0)),
            full, full, full,
            pl.BlockSpec((3, DIM), lambda i: (0, 0)), full,
            mat, full, mat, full,
            mat, mat, full,
        ],
        out_specs=[
            pl.BlockSpec((BLK, DIM), lambda i: (i, 0)),
            pl.BlockSpec((BLK, DIM), lambda i: (0, 0)),
            pl.BlockSpec((BLK, DIM), lambda i: (0, 0)),
        ],
        out_shape=[
            jax.ShapeDtypeStruct((N, DIM), jnp.float32),
            jax.ShapeDtypeStruct((SROWS, DIM), jnp.float32),
            jax.ShapeDtypeStruct((G, DIM), jnp.float32),
        ],
        scratch_shapes=[pltpu.VMEM((BLK, DIM), jnp.float32)],
    )(x, pos, jnp.asarray(_OM2), jnp.asarray(_SEL), jnp.asarray(_PH),
      proj_W, proj_b.reshape(1, DIM),
      pm_W1, pm_b1.reshape(1, DIM), pm_W2, pm_b2.reshape(1, DIM),
      W1a, W1b, g_bm1.reshape(1, DIM))


def _edges_body(ei_hbm, a_hbm, b_hbm, s_out, c_out,
                src_v, dst_v, csrc, cdst, sidx0, didx0, sidx1, didx1,
                arow0, brow0, arow1, brow1, ones_r, s_sp, c_sp, b_sp,
                sem_s0, sem_s1, sa0, sb0, sa1, sb1):
    c = lax.axis_index("c")
    s = lax.axis_index("s")
    wid = c * NS + s

    st0 = pltpu.async_copy(ei_hbm.at[pl.ds(wid * EPT, EPT)], src_v, sem_s0)
    st1 = pltpu.async_copy(ei_hbm.at[pl.ds(E + wid * EPT, EPT)], dst_v, sem_s1)

    @plsc.parallel_loop(0, BATCH, 1, unroll=2)
    def _fill(r):
        for k in range(DIM // LANES):
            arow0[r, pl.ds(k * LANES, LANES)] = jnp.zeros((LANES,), jnp.float32)
            ones_r[r, pl.ds(k * LANES, LANES)] = jnp.ones((LANES,), jnp.float32)
    pltpu.sync_copy(arow0.at[pl.ds(0, ZR)], s_sp.at[pl.ds(s * ZR, ZR)])
    pltpu.sync_copy(arow0.at[pl.ds(0, ZR)], c_sp.at[pl.ds(s * ZR, ZR)])
    WB = G // NS
    pltpu.sync_copy(b_hbm.at[pl.ds(s * WB, WB)], b_sp.at[pl.ds(s * WB, WB)])

    @pl.when(s == 0)
    def _():
        pltpu.sync_copy(b_hbm.at[pl.ds(G, SROWS - G)], b_sp.at[pl.ds(G, SROWS - G)])
    st0.wait()
    st1.wait()

    plsc.subcore_barrier()

    def _filt(i, offv):
        d = dst_v[pl.ds(i * LANES, LANES)]
        sv = src_v[pl.ds(i * LANES, LANES)]
        m = d < G
        idx = offv + plsc.cumsum(m.astype(jnp.int32)) - 1
        plsc.store_scatter(cdst, [idx], d, mask=m)
        plsc.store_scatter(csrc, [idx], sv, mask=m)
        return offv + plsc.all_reduce_population_count(m)
    offv = plsc.parallel_loop(0, EPT // LANES, 1, unroll=4,
                              carry=jnp.zeros((LANES,), jnp.int32))(_filt)
    n = jnp.sum(offv) // LANES

    for j in range(BATCH // LANES):
        cdst[pl.ds(n + j * LANES, LANES)] = jnp.full((LANES,), G, jnp.int32)
        csrc[pl.ds(n + j * LANES, LANES)] = jnp.zeros((LANES,), jnp.int32)
    nb = (n + BATCH - 1) // BATCH

    def _fire(b, sidx, didx, ar, br, sa, sb):
        for k in range(BATCH // LANES):
            sidx[pl.ds(k * LANES, LANES)] = csrc[pl.ds(b * BATCH + k * LANES, LANES)]
            didx[pl.ds(k * LANES, LANES)] = cdst[pl.ds(b * BATCH + k * LANES, LANES)]
        pltpu.async_copy(a_hbm.at[sidx], ar, sa)
        pltpu.async_copy(b_sp.at[didx], br, sb)

    def _wait(sidx, didx, ar, br, sa, sb):
        pltpu.make_async_copy(a_hbm.at[sidx], ar, sa).wait()
        pltpu.make_async_copy(b_sp.at[didx], br, sb).wait()

    def _compute_scat(didx, ar, br):
        @plsc.parallel_loop(0, BATCH, 1, unroll=4)
        def _row(r):
            for k in range(DIM // LANES):
                av = ar[r, pl.ds(k * LANES, LANES)]
                bv = br[r, pl.ds(k * LANES, LANES)]
                v = av + bv
                ar[r, pl.ds(k * LANES, LANES)] = v / (1.0 + jnp.exp(-v))
        pltpu.sync_copy(ar, s_sp.at[didx], add=True)
        pltpu.sync_copy(ones_r, c_sp.at[didx], add=True)

    @pl.when(nb > 0)
    def _():
        _fire(0, sidx0, didx0, arow0, brow0, sa0, sb0)

    @pl.when(nb > 1)
    def _():
        _fire(1, sidx1, didx1, arow1, brow1, sa1, sb1)

    def _pair(t, _):
        b0 = 2 * t
        _wait(sidx0, didx0, arow0, brow0, sa0, sb0)
        _compute_scat(didx0, arow0, brow0)

        @pl.when(b0 + 2 < nb)
        def _():
            _fire(b0 + 2, sidx0, didx0, arow0, brow0, sa0, sb0)

        @pl.when(b0 + 1 < nb)
        def _():
            _wait(sidx1, didx1, arow1, brow1, sa1, sb1)
            _compute_scat(didx1, arow1, brow1)

            @pl.when(b0 + 3 < nb)
            def _():
                _fire(b0 + 3, sidx1, didx1, arow1, brow1, sa1, sb1)
        return 0
    lax.fori_loop(0, (nb + 1) // 2, _pair, 0)

    plsc.subcore_barrier()

    WR = G // NS
    pltpu.sync_copy(s_sp.at[pl.ds(s * WR, WR)], s_out.at[c, pl.ds(s * WR, WR)])
    pltpu.sync_copy(c_sp.at[pl.ds(s * WR, WR)], c_out.at[c, pl.ds(s * WR, WR)])


def _edges(ei, A, Bpad):
    mesh = plsc.VectorSubcoreMesh(core_axis_name="c", subcore_axis_name="s")
    fn = pl.kernel(
        _edges_body,
        out_type=[
            jax.ShapeDtypeStruct((NC, G, DIM), jnp.float32),
            jax.ShapeDtypeStruct((NC, G, DIM), jnp.float32),
        ],
        mesh=mesh,
        compiler_params=pltpu.CompilerParams(needs_layout_passes=False),
        scratch_types=[
            pltpu.VMEM((EPT,), jnp.int32),
            pltpu.VMEM((EPT,), jnp.int32),
            pltpu.VMEM((CAP,), jnp.int32),
            pltpu.VMEM((CAP,), jnp.int32),
            pltpu.VMEM((BATCH,), jnp.int32),
            pltpu.VMEM((BATCH,), jnp.int32),
            pltpu.VMEM((BATCH,), jnp.int32),
            pltpu.VMEM((BATCH,), jnp.int32),
            pltpu.VMEM((BATCH, DIM), jnp.float32),
            pltpu.VMEM((BATCH, DIM), jnp.float32),
            pltpu.VMEM((BATCH, DIM), jnp.float32),
            pltpu.VMEM((BATCH, DIM), jnp.float32),
            pltpu.VMEM((BATCH, DIM), jnp.float32),
            pltpu.VMEM_SHARED((SROWS, DIM), jnp.float32),
            pltpu.VMEM_SHARED((SROWS, DIM), jnp.float32),
            pltpu.VMEM_SHARED((SROWS, DIM), jnp.float32),
            pltpu.SemaphoreType.DMA,
            pltpu.SemaphoreType.DMA,
            pltpu.SemaphoreType.DMA,
            pltpu.SemaphoreType.DMA,
            pltpu.SemaphoreType.DMA,
            pltpu.SemaphoreType.DMA,
        ],
    )
    return fn(ei, A, Bpad)


def _finish_body(hg_ref, s_ref, c_ref, w2_ref, b2_ref, o_ref):
    S = s_ref[0] + s_ref[1]
    C = c_ref[0] + c_ref[1]
    agg = jnp.dot(S, w2_ref[...], preferred_element_type=jnp.float32) + C * b2_ref[...]
    o_ref[...] = hg_ref[...] + agg / jnp.maximum(C, 1.0)


def _finish(hg, S2, C2, g_Wm2, g_bm2):
    return pl.pallas_call(
        _finish_body,
        out_shape=jax.ShapeDtypeStruct((G, DIM), jnp.float32),
    )(hg, S2, C2, g_Wm2, g_bm2.reshape(1, DIM))


def kernel(x, pos, batch_idx, edge_index, proj_W, proj_b,
           pm_W1, pm_b1, pm_W2, pm_b2, g_Wm1, g_bm1, g_Wm2, g_bm2):
    ei = edge_index.astype(jnp.int32).reshape(2 * E)
    W1a = g_Wm1[:DIM]
    W1b = g_Wm1[DIM:]
    A, Bpad, hg = _prep(x, pos, proj_W, proj_b,
                        pm_W1, pm_b1, pm_W2, pm_b2, W1a, W1b, g_bm1)
    S2, C2 = _edges(ei, A, Bpad)
    out = _finish(hg, S2, C2, g_Wm2, g_bm2)
    return out.reshape(1, G, DIM)


# speedup vs baseline: 1.0185x; 1.0185x over previous
import numpy as np
import jax
import jax.numpy as jnp
from jax import lax
from jax.experimental import pallas as pl
from jax.experimental.pallas import tpu as pltpu
from jax.experimental.pallas import tpu_sc as plsc

N = 10000
E = 320000
DIM = 128
G = 1024
BLK = 1024
NBLK = (N + BLK - 1) // BLK

NC = 2
NS = 16
NW = NC * NS
LANES = 16
EPT = E // NW
BATCH = 96
CAP = EPT + 2 * BATCH
SROWS = G + LANES
CW = 16
ZR = SROWS // NS

_half = 32
_om = 1.0 / (10000.0 ** (np.arange(_half, dtype=np.float32) / _half))
_OM2 = np.concatenate([_om, _om, _om, _om]).reshape(1, DIM).astype(np.float32)
_SEL = np.concatenate([np.zeros(64), np.ones(64)]).reshape(1, DIM).astype(np.float32)
_PH = np.concatenate([np.zeros(32), np.full(32, np.pi / 2),
                      np.zeros(32), np.full(32, np.pi / 2)]).reshape(1, DIM)
_PH = _PH.astype(np.float32)


def _silu(v):
    return v * (1.0 / (1.0 + jnp.exp(-v)))


def _prep_body(x_ref, pos_ref, om_ref, sel_ref, ph_ref,
               pW_ref, pb_ref, w1_ref, b1_ref, w2_ref, b2_ref,
               wa_ref, wb_ref, gb1_ref,
               a_ref, b_ref, hg_ref, h_s):
    pid = pl.program_id(0)
    x = x_ref[...]
    pos = pos_ref[...]
    sel = sel_ref[...]
    posc = pos[:, 0:1] * (1.0 - sel) + pos[:, 1:2] * sel
    pe = jnp.sin(posc * om_ref[...] + ph_ref[...])

    @pl.when(pid < 1)
    def _():
        t = _silu(jnp.dot(pe, w1_ref[...], preferred_element_type=jnp.float32)
                  + b1_ref[...])
        u = (jnp.dot(t, w2_ref[...], preferred_element_type=jnp.float32)
             + b2_ref[...])
        h = u + pe
        h_s[...] = h
        b_ref[...] = (jnp.dot(h, wb_ref[...], preferred_element_type=jnp.float32)
                      + gb1_ref[...])
        hg_ref[...] = h

    @pl.when(pid >= 1)
    def _():
        h_s[...] = (x[:, 0:1] * pW_ref[0:1, :] + x[:, 1:2] * pW_ref[1:2, :]
                    + x[:, 2:3] * pW_ref[2:3, :] + pb_ref[...]) + pe

    a_ref[...] = jnp.dot(h_s[...], wa_ref[...],
                         preferred_element_type=jnp.float32)


def _prep(x, pos, proj_W, proj_b, pm_W1, pm_b1, pm_W2, pm_b2, W1a, W1b, g_bm1):
    full = pl.BlockSpec((1, DIM), lambda i: (0, 0))
    mat = pl.BlockSpec((DIM, DIM), lambda i: (0, 0))
    return pl.pallas_call(
        _prep_body,
        grid=(NBLK,),
        in_specs=[
            pl.BlockSpec((BLK, 3), lambda i: (i, 0)),
            pl.BlockSpec((BLK, 2), lambda i: (i, 0)),
            full, full, full,
            pl.BlockSpec((3, DIM), lambda i: (0, 0)), full,
            mat, full, mat, full,
            mat, mat, full,
        ],
        out_specs=[
            pl.BlockSpec((BLK, DIM), lambda i: (i, 0)),
            pl.BlockSpec((BLK, DIM), lambda i: (0, 0)),
            pl.BlockSpec((BLK, DIM), lambda i: (0, 0)),
        ],
        out_shape=[
            jax.ShapeDtypeStruct((N, DIM), jnp.float32),
            jax.ShapeDtypeStruct((SROWS, DIM), jnp.float32),
            jax.ShapeDtypeStruct((G, DIM), jnp.float32),
        ],
        scratch_shapes=[pltpu.VMEM((BLK, DIM), jnp.float32)],
    )(x, pos, jnp.asarray(_OM2), jnp.asarray(_SEL), jnp.asarray(_PH),
      proj_W, proj_b.reshape(1, DIM),
      pm_W1, pm_b1.reshape(1, DIM), pm_W2, pm_b2.reshape(1, DIM),
      W1a, W1b, g_bm1.reshape(1, DIM))


def _edges_body(ei_hbm, a_hbm, b_hbm, s_out, c_out,
                src_v, dst_v, csrc, cdst, sidx0, didx0, sidx1, didx1,
                arow0, brow0, arow1, brow1, ones_r, s_sp, c_sp, b_sp,
                sem_s0, sem_s1, sa0, sb0, sa1, sb1):
    c = lax.axis_index("c")
    s = lax.axis_index("s")
    wid = c * NS + s

    st0 = pltpu.async_copy(ei_hbm.at[pl.ds(wid * EPT, EPT)], src_v, sem_s0)
    st1 = pltpu.async_copy(ei_hbm.at[pl.ds(E + wid * EPT, EPT)], dst_v, sem_s1)

    @plsc.parallel_loop(0, BATCH, 1, unroll=2)
    def _fill(r):
        for k in range(DIM // LANES):
            arow0[r, pl.ds(k * LANES, LANES)] = jnp.zeros((LANES,), jnp.float32)
            ones_r[r, pl.ds(k * LANES, LANES)] = jnp.ones((LANES,), jnp.float32)
    pltpu.sync_copy(arow0.at[pl.ds(0, ZR)], s_sp.at[pl.ds(s * ZR, ZR)])
    pltpu.sync_copy(arow0.at[pl.ds(0, ZR)], c_sp.at[pl.ds(s * ZR, ZR)])
    WB = G // NS
    pltpu.sync_copy(b_hbm.at[pl.ds(s * WB, WB)], b_sp.at[pl.ds(s * WB, WB)])

    @pl.when(s == 0)
    def _():
        pltpu.sync_copy(b_hbm.at[pl.ds(G, SROWS - G)], b_sp.at[pl.ds(G, SROWS - G)])
    st0.wait()
    st1.wait()

    plsc.subcore_barrier()

    def _filt(i, offv):
        d = dst_v[pl.ds(i * LANES, LANES)]
        sv = src_v[pl.ds(i * LANES, LANES)]
        m = d < G
        idx = offv + plsc.cumsum(m.astype(jnp.int32)) - 1
        plsc.store_scatter(cdst, [idx], d, mask=m)
        plsc.store_scatter(csrc, [idx], sv, mask=m)
        return offv + plsc.all_reduce_population_count(m)
    offv = plsc.parallel_loop(0, EPT // LANES, 1, unroll=4,
                              carry=jnp.zeros((LANES,), jnp.int32))(_filt)
    n = jnp.sum(offv) // LANES

    for j in range(BATCH // LANES):
        cdst[pl.ds(n + j * LANES, LANES)] = jnp.full((LANES,), G, jnp.int32)
        csrc[pl.ds(n + j * LANES, LANES)] = jnp.zeros((LANES,), jnp.int32)
    nb = (n + BATCH - 1) // BATCH

    def _fire(b, sidx, didx, ar, br, sa, sb):
        for k in range(BATCH // LANES):
            sidx[pl.ds(k * LANES, LANES)] = csrc[pl.ds(b * BATCH + k * LANES, LANES)]
            didx[pl.ds(k * LANES, LANES)] = cdst[pl.ds(b * BATCH + k * LANES, LANES)]
        pltpu.async_copy(a_hbm.at[sidx], ar, sa)
        pltpu.async_copy(b_sp.at[didx], br, sb)

    def _wait(sidx, didx, ar, br, sa, sb):
        pltpu.make_async_copy(a_hbm.at[sidx], ar, sa).wait()
        pltpu.make_async_copy(b_sp.at[didx], br, sb).wait()

    def _compute_scat(didx, ar, br):
        @plsc.parallel_loop(0, BATCH, 1, unroll=2)
        def _row(r):
            for k in range(DIM // LANES):
                av = ar[r, pl.ds(k * LANES, LANES)]
                bv = br[r, pl.ds(k * LANES, LANES)]
                v = av + bv
                ar[r, pl.ds(k * LANES, LANES)] = v / (1.0 + jnp.exp(-v))
        pltpu.sync_copy(ar, s_sp.at[didx], add=True)
        pltpu.sync_copy(ones_r, c_sp.at[didx], add=True)

    @pl.when(nb > 0)
    def _():
        _fire(0, sidx0, didx0, arow0, brow0, sa0, sb0)

    @pl.when(nb > 1)
    def _():
        _fire(1, sidx1, didx1, arow1, brow1, sa1, sb1)

    def _pair(t, _):
        b0 = 2 * t
        _wait(sidx0, didx0, arow0, brow0, sa0, sb0)
        _compute_scat(didx0, arow0, brow0)

        @pl.when(b0 + 2 < nb)
        def _():
            _fire(b0 + 2, sidx0, didx0, arow0, brow0, sa0, sb0)

        @pl.when(b0 + 1 < nb)
        def _():
            _wait(sidx1, didx1, arow1, brow1, sa1, sb1)
            _compute_scat(didx1, arow1, brow1)

            @pl.when(b0 + 3 < nb)
            def _():
                _fire(b0 + 3, sidx1, didx1, arow1, brow1, sa1, sb1)
        return 0
    lax.fori_loop(0, (nb + 1) // 2, _pair, 0)

    plsc.subcore_barrier()

    WR = G // NS
    pltpu.sync_copy(s_sp.at[pl.ds(s * WR, WR)], s_out.at[c, pl.ds(s * WR, WR)])
    pltpu.sync_copy(c_sp.at[pl.ds(s * WR, WR)], c_out.at[c, pl.ds(s * WR, WR)])


def _edges(ei, A, Bpad):
    mesh = plsc.VectorSubcoreMesh(core_axis_name="c", subcore_axis_name="s")
    fn = pl.kernel(
        _edges_body,
        out_type=[
            jax.ShapeDtypeStruct((NC, G, DIM), jnp.float32),
            jax.ShapeDtypeStruct((NC, G, DIM), jnp.float32),
        ],
        mesh=mesh,
        compiler_params=pltpu.CompilerParams(needs_layout_passes=False),
        scratch_types=[
            pltpu.VMEM((EPT,), jnp.int32),
            pltpu.VMEM((EPT,), jnp.int32),
            pltpu.VMEM((CAP,), jnp.int32),
            pltpu.VMEM((CAP,), jnp.int32),
            pltpu.VMEM((BATCH,), jnp.int32),
            pltpu.VMEM((BATCH,), jnp.int32),
            pltpu.VMEM((BATCH,), jnp.int32),
            pltpu.VMEM((BATCH,), jnp.int32),
            pltpu.VMEM((BATCH, DIM), jnp.float32),
            pltpu.VMEM((BATCH, DIM), jnp.float32),
            pltpu.VMEM((BATCH, DIM), jnp.float32),
            pltpu.VMEM((BATCH, DIM), jnp.float32),
            pltpu.VMEM((BATCH, DIM), jnp.float32),
            pltpu.VMEM_SHARED((SROWS, DIM), jnp.float32),
            pltpu.VMEM_SHARED((SROWS, DIM), jnp.float32),
            pltpu.VMEM_SHARED((SROWS, DIM), jnp.float32),
            pltpu.SemaphoreType.DMA,
            pltpu.SemaphoreType.DMA,
            pltpu.SemaphoreType.DMA,
            pltpu.SemaphoreType.DMA,
            pltpu.SemaphoreType.DMA,
            pltpu.SemaphoreType.DMA,
        ],
    )
    return fn(ei, A, Bpad)


def _finish_body(hg_ref, s_ref, c_ref, w2_ref, b2_ref, o_ref):
    S = s_ref[0] + s_ref[1]
    C = c_ref[0] + c_ref[1]
    agg = jnp.dot(S, w2_ref[...], preferred_element_type=jnp.float32) + C * b2_ref[...]
    o_ref[...] = hg_ref[...] + agg / jnp.maximum(C, 1.0)


def _finish(hg, S2, C2, g_Wm2, g_bm2):
    return pl.pallas_call(
        _finish_body,
        out_shape=jax.ShapeDtypeStruct((G, DIM), jnp.float32),
    )(hg, S2, C2, g_Wm2, g_bm2.reshape(1, DIM))


def kernel(x, pos, batch_idx, edge_index, proj_W, proj_b,
           pm_W1, pm_b1, pm_W2, pm_b2, g_Wm1, g_bm1, g_Wm2, g_bm2):
    ei = edge_index.astype(jnp.int32).reshape(2 * E)
    W1a = g_Wm1[:DIM]
    W1b = g_Wm1[DIM:]
    A, Bpad, hg = _prep(x, pos, proj_W, proj_b,
                        pm_W1, pm_b1, pm_W2, pm_b2, W1a, W1b, g_bm1)
    S2, C2 = _edges(ei, A, Bpad)
    out = _finish(hg, S2, C2, g_Wm2, g_bm2)
    return out.reshape(1, G, DIM)


# speedup vs baseline: 1.0202x; 1.0016x over previous
import numpy as np
import jax
import jax.numpy as jnp
from jax import lax
from jax.experimental import pallas as pl
from jax.experimental.pallas import tpu as pltpu
from jax.experimental.pallas import tpu_sc as plsc

N = 10000
E = 320000
DIM = 128
G = 1024
BLK = 1024
NBLK = (N + BLK - 1) // BLK

NC = 2
NS = 16
NW = NC * NS
LANES = 16
EPT = E // NW
BATCH = 96
CAP = EPT + 2 * BATCH
SROWS = G + LANES
ZR = SROWS // NS

_half = 32
_om = 1.0 / (10000.0 ** (np.arange(_half, dtype=np.float32) / _half))
_OM2 = np.concatenate([_om, _om, _om, _om]).reshape(1, DIM).astype(np.float32)
_SEL = np.concatenate([np.zeros(64), np.ones(64)]).reshape(1, DIM).astype(np.float32)
_PH = np.concatenate([np.zeros(32), np.full(32, np.pi / 2),
                      np.zeros(32), np.full(32, np.pi / 2)]).reshape(1, DIM)
_PH = _PH.astype(np.float32)


def _silu(v):
    return v * (1.0 / (1.0 + jnp.exp(-v)))


def _prep_body(x_ref, pos_ref, om_ref, sel_ref, ph_ref,
               pW_ref, pb_ref, w1_ref, b1_ref, w2_ref, b2_ref,
               wa_ref, wb_ref, gb1_ref,
               a_ref, b_ref, hg_ref, h_s):
    pid = pl.program_id(0)
    x = x_ref[...]
    pos = pos_ref[...]
    sel = sel_ref[...]
    posc = pos[:, 0:1] * (1.0 - sel) + pos[:, 1:2] * sel
    pe = jnp.sin(posc * om_ref[...] + ph_ref[...])

    @pl.when(pid < 1)
    def _():
        t = _silu(jnp.dot(pe, w1_ref[...], preferred_element_type=jnp.float32)
                  + b1_ref[...])
        u = (jnp.dot(t, w2_ref[...], preferred_element_type=jnp.float32)
             + b2_ref[...])
        h = u + pe
        h_s[...] = h
        b_ref[...] = (jnp.dot(h, wb_ref[...], preferred_element_type=jnp.float32)
                      + gb1_ref[...])
        hg_ref[...] = h

    @pl.when(pid >= 1)
    def _():
        h_s[...] = (x[:, 0:1] * pW_ref[0:1, :] + x[:, 1:2] * pW_ref[1:2, :]
                    + x[:, 2:3] * pW_ref[2:3, :] + pb_ref[...]) + pe

    a_ref[...] = jnp.dot(h_s[...], wa_ref[...],
                         preferred_element_type=jnp.float32)


def _prep(x, pos, proj_W, proj_b, pm_W1, pm_b1, pm_W2, pm_b2, W1a, W1b, g_bm1):
    full = pl.BlockSpec((1, DIM), lambda i: (0, 0))
    mat = pl.BlockSpec((DIM, DIM), lambda i: (0, 0))
    return pl.pallas_call(
        _prep_body,
        grid=(NBLK,),
        in_specs=[
            pl.BlockSpec((BLK, 3), lambda i: (i, 0)),
            pl.BlockSpec((BLK, 2), lambda i: (i, 0)),
            full, full, full,
            pl.BlockSpec((3, DIM), lambda i: (0, 0)), full,
            mat, full, mat, full,
            mat, mat, full,
        ],
        out_specs=[
            pl.BlockSpec((BLK, DIM), lambda i: (i, 0)),
            pl.BlockSpec((BLK, DIM), lambda i: (0, 0)),
            pl.BlockSpec((BLK, DIM), lambda i: (0, 0)),
        ],
        out_shape=[
            jax.ShapeDtypeStruct((N, DIM), jnp.float32),
            jax.ShapeDtypeStruct((SROWS, DIM), jnp.float32),
            jax.ShapeDtypeStruct((G, DIM), jnp.float32),
        ],
        scratch_shapes=[pltpu.VMEM((BLK, DIM), jnp.float32)],
    )(x, pos, jnp.asarray(_OM2), jnp.asarray(_SEL), jnp.asarray(_PH),
      proj_W, proj_b.reshape(1, DIM),
      pm_W1, pm_b1.reshape(1, DIM), pm_W2, pm_b2.reshape(1, DIM),
      W1a, W1b, g_bm1.reshape(1, DIM))


def _edges_body(ei_hbm, a_hbm, b_hbm, s_out, c_out,
                src_v, dst_v, csrc, cdst, sidx0, didx0, sidx1, didx1,
                arow0, brow0, arow1, brow1, ones_r, s_sp, c_sp, b_sp,
                sem_s0, sem_s1, sa0, sb0, sa1, sb1):
    c = lax.axis_index("c")
    s = lax.axis_index("s")
    wid = c * NS + s

    st0 = pltpu.async_copy(ei_hbm.at[pl.ds(wid * EPT, EPT)], src_v, sem_s0)
    st1 = pltpu.async_copy(ei_hbm.at[pl.ds(E + wid * EPT, EPT)], dst_v, sem_s1)

    @plsc.parallel_loop(0, BATCH, 1, unroll=2)
    def _fill(r):
        for k in range(DIM // LANES):
            arow0[r, pl.ds(k * LANES, LANES)] = jnp.zeros((LANES,), jnp.float32)
            ones_r[r, pl.ds(k * LANES, LANES)] = jnp.ones((LANES,), jnp.float32)
    pltpu.sync_copy(arow0.at[pl.ds(0, ZR)], s_sp.at[pl.ds(s * ZR, ZR)])
    pltpu.sync_copy(arow0.at[pl.ds(0, ZR)], c_sp.at[pl.ds(s * ZR, ZR)])
    WB = G // NS
    pltpu.sync_copy(b_hbm.at[pl.ds(s * WB, WB)], b_sp.at[pl.ds(s * WB, WB)])

    @pl.when(s == 0)
    def _():
        pltpu.sync_copy(b_hbm.at[pl.ds(G, SROWS - G)], b_sp.at[pl.ds(G, SROWS - G)])
    st0.wait()
    st1.wait()

    plsc.subcore_barrier()

    def _filt(i, offv):
        d = dst_v[pl.ds(i * LANES, LANES)]
        sv = src_v[pl.ds(i * LANES, LANES)]
        m = d < G
        idx = offv + plsc.cumsum(m.astype(jnp.int32)) - 1
        plsc.store_scatter(cdst, [idx], d, mask=m)
        plsc.store_scatter(csrc, [idx], sv, mask=m)
        return offv + plsc.all_reduce_population_count(m)
    offv = plsc.parallel_loop(0, EPT // LANES, 1, unroll=4,
                              carry=jnp.zeros((LANES,), jnp.int32))(_filt)
    n = jnp.sum(offv) // LANES

    for j in range(BATCH // LANES):
        cdst[pl.ds(n + j * LANES, LANES)] = jnp.full((LANES,), G, jnp.int32)
        csrc[pl.ds(n + j * LANES, LANES)] = jnp.zeros((LANES,), jnp.int32)
    nb = (n + BATCH - 1) // BATCH

    def _fire(b, sidx, didx, ar, br, sa, sb):
        for k in range(BATCH // LANES):
            sidx[pl.ds(k * LANES, LANES)] = csrc[pl.ds(b * BATCH + k * LANES, LANES)]
            didx[pl.ds(k * LANES, LANES)] = cdst[pl.ds(b * BATCH + k * LANES, LANES)]
        pltpu.async_copy(a_hbm.at[sidx], ar, sa)
        pltpu.async_copy(b_sp.at[didx], br, sb)

    def _wait(sidx, didx, ar, br, sa, sb):
        pltpu.make_async_copy(a_hbm.at[sidx], ar, sa).wait()
        pltpu.make_async_copy(b_sp.at[didx], br, sb).wait()

    def _compute_scat(didx, ar, br):
        @plsc.parallel_loop(0, BATCH, 1, unroll=2)
        def _row(r):
            for k in range(DIM // LANES):
                av = ar[r, pl.ds(k * LANES, LANES)]
                bv = br[r, pl.ds(k * LANES, LANES)]
                v = av + bv
                ar[r, pl.ds(k * LANES, LANES)] = v / (1.0 + jnp.exp(-v))
        pltpu.sync_copy(ar, s_sp.at[didx], add=True)
        pltpu.sync_copy(ones_r, c_sp.at[didx], add=True)

    @pl.when(nb > 0)
    def _():
        _fire(0, sidx0, didx0, arow0, brow0, sa0, sb0)

    @pl.when(nb > 1)
    def _():
        _fire(1, sidx1, didx1, arow1, brow1, sa1, sb1)

    def _pair(t, _):
        b0 = 2 * t
        _wait(sidx0, didx0, arow0, brow0, sa0, sb0)
        _compute_scat(didx0, arow0, brow0)

        @pl.when(b0 + 2 < nb)
        def _():
            _fire(b0 + 2, sidx0, didx0, arow0, brow0, sa0, sb0)

        @pl.when(b0 + 1 < nb)
        def _():
            _wait(sidx1, didx1, arow1, brow1, sa1, sb1)
            _compute_scat(didx1, arow1, brow1)

            @pl.when(b0 + 3 < nb)
            def _():
                _fire(b0 + 3, sidx1, didx1, arow1, brow1, sa1, sb1)
        return 0
    lax.fori_loop(0, (nb + 1) // 2, _pair, 0)

    plsc.subcore_barrier()

    WR = G // NS
    pltpu.sync_copy(s_sp.at[pl.ds(s * WR, WR)], s_out.at[c, pl.ds(s * WR, WR)])
    pltpu.sync_copy(c_sp.at[pl.ds(s * WR, WR)], c_out.at[c, pl.ds(s * WR, WR)])


def _edges(ei, A, Bpad):
    mesh = plsc.VectorSubcoreMesh(core_axis_name="c", subcore_axis_name="s")
    fn = pl.kernel(
        _edges_body,
        out_type=[
            jax.ShapeDtypeStruct((NC, G, DIM), jnp.float32),
            jax.ShapeDtypeStruct((NC, G, DIM), jnp.float32),
        ],
        mesh=mesh,
        compiler_params=pltpu.CompilerParams(needs_layout_passes=False),
        scratch_types=[
            pltpu.VMEM((EPT,), jnp.int32),
            pltpu.VMEM((EPT,), jnp.int32),
            pltpu.VMEM((CAP,), jnp.int32),
            pltpu.VMEM((CAP,), jnp.int32),
            pltpu.VMEM((BATCH,), jnp.int32),
            pltpu.VMEM((BATCH,), jnp.int32),
            pltpu.VMEM((BATCH,), jnp.int32),
            pltpu.VMEM((BATCH,), jnp.int32),
            pltpu.VMEM((BATCH, DIM), jnp.float32),
            pltpu.VMEM((BATCH, DIM), jnp.float32),
            pltpu.VMEM((BATCH, DIM), jnp.float32),
            pltpu.VMEM((BATCH, DIM), jnp.float32),
            pltpu.VMEM((BATCH, DIM), jnp.float32),
            pltpu.VMEM_SHARED((SROWS, DIM), jnp.float32),
            pltpu.VMEM_SHARED((SROWS, DIM), jnp.float32),
            pltpu.VMEM_SHARED((SROWS, DIM), jnp.float32),
            pltpu.SemaphoreType.DMA,
            pltpu.SemaphoreType.DMA,
            pltpu.SemaphoreType.DMA,
            pltpu.SemaphoreType.DMA,
            pltpu.SemaphoreType.DMA,
            pltpu.SemaphoreType.DMA,
        ],
    )
    return fn(ei, A, Bpad)


def _finish_body(hg_ref, s_ref, c_ref, w2_ref, b2_ref, o_ref):
    S = s_ref[0] + s_ref[1]
    C = c_ref[0] + c_ref[1]
    agg = jnp.dot(S, w2_ref[...], preferred_element_type=jnp.float32) + C * b2_ref[...]
    o_ref[...] = hg_ref[...] + agg / jnp.maximum(C, 1.0)


def _finish(hg, S2, C2, g_Wm2, g_bm2):
    return pl.pallas_call(
        _finish_body,
        out_shape=jax.ShapeDtypeStruct((G, DIM), jnp.float32),
    )(hg, S2, C2, g_Wm2, g_bm2.reshape(1, DIM))


def kernel(x, pos, batch_idx, edge_index, proj_W, proj_b,
           pm_W1, pm_b1, pm_W2, pm_b2, g_Wm1, g_bm1, g_Wm2, g_bm2):
    ei = edge_index.astype(jnp.int32).reshape(2 * E)
    W1a = g_Wm1[:DIM]
    W1b = g_Wm1[DIM:]
    A, Bpad, hg = _prep(x, pos, proj_W, proj_b,
                        pm_W1, pm_b1, pm_W2, pm_b2, W1a, W1b, g_bm1)
    S2, C2 = _edges(ei, A, Bpad)
    out = _finish(hg, S2, C2, g_Wm2, g_bm2)
    return out.reshape(1, G, DIM)
